# baseline (speedup 1.0000x reference)
import functools

import jax
import jax.numpy as jnp
from jax import lax
from jax.experimental import pallas as pl
from jax.experimental.pallas import tpu as pltpu
from jax.experimental.pallas import tpu_sc as plsc

_NX, _NY, _NZ = 128, 128, 128
_SX, _SY, _SZ = 2.0, 2.0, 2.0
_CX, _CY, _CZ = 0.0, 0.0, 0.0
_SCALE_MOD = 1.0
_HX, _HY, _HZ = _SX / _NX, _SY / _NY, _SZ / _NZ
_GX = _CX - _SX / 2 + 0.5 * _HX
_GY = _CY - _SY / 2 + 0.5 * _HY
_GZ = _CZ - _SZ / 2 + 0.5 * _HZ
_NVOX = _NX * _NY * _NZ
_HALF = _NVOX // 2

_BG_SUB = 32
_BG = _BG_SUB * 128
_OFF = (-1, 0, 1)


def _gauss_body(p_ref, val_ref, idx_ref, rad_ref):
  mx, my, mz = p_ref[0], p_ref[1], p_ref[2]
  sx = p_ref[3] * _SCALE_MOD
  sy = p_ref[4] * _SCALE_MOD
  sz = p_ref[5] * _SCALE_MOD
  qw, qx, qy, qz = p_ref[6], p_ref[7], p_ref[8], p_ref[9]
  op = p_ref[10]

  rmax = jnp.maximum(sx, jnp.maximum(sy, sz))
  rad_ref[...] = jnp.ceil(3.0 * rmax / (_SX / _NX)).astype(jnp.int32)

  nrm = jnp.sqrt(qw * qw + qx * qx + qy * qy + qz * qz) + 1e-8
  w, x, y, z = qw / nrm, qx / nrm, qy / nrm, qz / nrm
  r00 = 1 - 2 * (y * y + z * z)
  r01 = 2 * (x * y - w * z)
  r02 = 2 * (x * z + w * y)
  r10 = 2 * (x * y + w * z)
  r11 = 1 - 2 * (x * x + z * z)
  r12 = 2 * (y * z - w * x)
  r20 = 2 * (x * z - w * y)
  r21 = 2 * (y * z + w * x)
  r22 = 1 - 2 * (x * x + y * y)

  s0, s1, s2 = sx * sx, sy * sy, sz * sz
  a = r00 * r00 * s0 + r01 * r01 * s1 + r02 * r02 * s2 + 1e-8
  b = r00 * r10 * s0 + r01 * r11 * s1 + r02 * r12 * s2
  c = r00 * r20 * s0 + r01 * r21 * s1 + r02 * r22 * s2
  d = r10 * r10 * s0 + r11 * r11 * s1 + r12 * r12 * s2 + 1e-8
  e = r10 * r20 * s0 + r11 * r21 * s1 + r12 * r22 * s2
  f = r20 * r20 * s0 + r21 * r21 * s1 + r22 * r22 * s2 + 1e-8

  c00 = d * f - e * e
  c01 = c * e - b * f
  c02 = b * e - c * d
  c11 = a * f - c * c
  c12 = b * c - a * e
  c22 = a * d - b * b
  det = a * c00 + b * c01 + c * c02
  inv = 1.0 / det
  i00, i11, i22 = c00 * inv, c11 * inv, c22 * inv
  i01_2, i02_2, i12_2 = 2 * c01 * inv, 2 * c02 * inv, 2 * c12 * inv

  bxf = jnp.round((mx - _GX) / _HX)
  byf = jnp.round((my - _GY) / _HY)
  bzf = jnp.round((mz - _GZ) / _HZ)
  bxi = bxf.astype(jnp.int32)
  byi = byf.astype(jnp.int32)
  bzi = bzf.astype(jnp.int32)

  dxs, dys, dzs = {}, {}, {}
  vxm, vym, vzm = {}, {}, {}
  xc, yc, zc = {}, {}, {}
  for o in _OFF:
    of = float(o)
    dxs[o] = (_GX + (bxf + of) * _HX) - mx
    dys[o] = (_GY + (byf + of) * _HY) - my
    dzs[o] = (_GZ + (bzf + of) * _HZ) - mz
    ix, iy, iz = bxi + o, byi + o, bzi + o
    vxm[o] = (ix >= 0) & (ix < _NX)
    vym[o] = (iy >= 0) & (iy < _NY)
    vzm[o] = (iz >= 0) & (iz < _NZ)
    xc[o] = jnp.clip(ix, 0, _NX - 1) * (_NY * _NZ)
    yc[o] = jnp.clip(iy, 0, _NY - 1) * _NZ
    zc[o] = jnp.clip(iz, 0, _NZ - 1)

  i22dz = {o: i22 * dzs[o] for o in _OFF}
  k = 0
  for ox in _OFF:
    dx = dxs[ox]
    hx1 = i01_2 * dx
    hx2 = i02_2 * dx
    qx_part = i00 * dx * dx
    for oy in _OFF:
      dy = dys[oy]
      qxy = qx_part + dy * (i11 * dy + hx1)
      t2 = i12_2 * dy + hx2
      vxy = vxm[ox] & vym[oy]
      fxy = xc[ox] + yc[oy]
      for oz in _OFF:
        dz = dzs[oz]
        q = qxy + dz * (i22dz[oz] + t2)
        opv = jnp.where(vxy & vzm[oz], op, 0.0)
        alpha = opv * jnp.exp(-0.5 * q)
        val_ref[k] = alpha
        idx_ref[k] = fxy + zc[oz]
        k += 1


def _sc_scatter(valf, idx2, zeros):
  rows_total = idx2.shape[0]
  rows_tile = rows_total // 16
  wrows = 72
  assert rows_tile % wrows == 0 and rows_tile % 8 == 0
  nwin = rows_tile // wrows
  win = wrows * 128
  seg = _HALF // 16
  mesh = plsc.VectorSubcoreMesh(core_axis_name="c", subcore_axis_name="s")

  @functools.partial(
      pl.kernel,
      mesh=mesh,
      out_type=jax.ShapeDtypeStruct((_NVOX,), jnp.float32),
      scratch_types=[
          pltpu.VMEM_SHARED((_HALF,), jnp.float32),
          pltpu.VMEM((wrows, 128), jnp.int32),
          pltpu.VMEM((wrows, 128), jnp.int32),
          pltpu.VMEM((win,), jnp.int32),
          pltpu.VMEM((win,), jnp.int32),
          pltpu.VMEM((win,), jnp.float32),
          pltpu.VMEM((win,), jnp.float32),
          pltpu.SemaphoreType.DMA,
          pltpu.SemaphoreType.DMA,
          pltpu.SemaphoreType.DMA,
          pltpu.SemaphoreType.DMA,
      ],
  )
  def scatter_kernel(val_hbm, idx_hbm, zero_hbm, out_hbm, grid_sp, idx_a,
                     idx_b, idxt_a, idxt_b, val_a, val_b, sem_la, sem_lb,
                     sem_sa, sem_sb):
    cid = lax.axis_index("c")
    sid = lax.axis_index("s")
    pltpu.sync_copy(zero_hbm.at[pl.ds(sid * seg, seg)],
                    grid_sp.at[pl.ds(sid * seg, seg)])
    plsc.subcore_barrier()
    base = sid * rows_tile

    idx_bufs = (idx_a, idx_b)
    idxt_bufs = (idxt_a, idxt_b)
    val_bufs = (val_a, val_b)
    sem_l = (sem_la, sem_lb)
    sem_s = (sem_sa, sem_sb)

    c_lo = cid * _HALF

    def start_loads(w, bb):
      r0 = base + w * wrows
      hi = pltpu.async_copy(idx_hbm.at[pl.ds(r0, wrows), :], idx_bufs[bb],
                            sem_l[bb])
      hv = pltpu.async_copy(val_hbm.at[pl.ds(r0 * 128, win)], val_bufs[bb],
                            sem_l[bb])
      return hi, hv

    def transform(bb):
      src = idx_bufs[bb]
      dst = idxt_bufs[bb]

      def tbody(r, carry):
        for cc in range(8):
          v = src[r, pl.ds(cc * 16, 16)]
          t = v - c_lo
          u = plsc.bitcast(t, jnp.uint32)
          dst[pl.ds(r * 128 + cc * 16, 16)] = jnp.where(
              u < jnp.uint32(_HALF), t, -1)
        return carry

      lax.fori_loop(0, wrows, tbody, 0)

    loads = [None, None]
    scats = [None, None]
    loads[0] = start_loads(0, 0)
    for wi in range(nwin):
      cur = wi & 1
      nxt = 1 - cur
      loads[cur][0].wait()
      loads[cur][1].wait()
      if wi + 1 < nwin:
        if scats[nxt] is not None:
          scats[nxt].wait()
        loads[nxt] = start_loads(wi + 1, nxt)
      transform(cur)
      scats[cur] = pltpu.async_copy(
          val_bufs[cur],
          grid_sp.at[plsc.Indices(idxt_bufs[cur], ignored_value=-1)],
          sem_s[cur],
          add=True,
      )
    for bb in range(2):
      if scats[bb] is not None:
        scats[bb].wait()
    plsc.subcore_barrier()
    pltpu.sync_copy(grid_sp.at[pl.ds(sid * seg, seg)],
                    out_hbm.at[pl.ds(cid * _HALF + sid * seg, seg)])

  return scatter_kernel(valf, idx2, zeros)


def kernel(means3D, opacities, scales, rotations):
  n = means3D.shape[0]
  nb = -(-n // _BG)
  nb = -(-nb // 4) * 4
  npad = nb * _BG
  sub = npad // 128
  pad = npad - n

  def padn(arr, v):
    return jnp.pad(arr, ((0, 0), (0, pad)), constant_values=v)

  params = jnp.concatenate([
      padn(means3D.T, 0.0),
      padn(scales.T, 0.01),
      padn(rotations.T, 1.0),
      padn(opacities.T, 0.0),
  ], axis=0).reshape(11, sub, 128)

  val, idx, rad = pl.pallas_call(
      _gauss_body,
      grid=(nb,),
      in_specs=[pl.BlockSpec((11, _BG_SUB, 128), lambda i: (0, i, 0))],
      out_specs=[
          pl.BlockSpec((27, _BG_SUB, 128), lambda i: (0, i, 0)),
          pl.BlockSpec((27, _BG_SUB, 128), lambda i: (0, i, 0)),
          pl.BlockSpec((_BG_SUB, 128), lambda i: (i, 0)),
      ],
      out_shape=[
          jax.ShapeDtypeStruct((27, sub, 128), jnp.float32),
          jax.ShapeDtypeStruct((27, sub, 128), jnp.int32),
          jax.ShapeDtypeStruct((sub, 128), jnp.int32),
      ],
  )(params)

  rows_total = 27 * sub
  zeros = jnp.zeros((_HALF,), jnp.float32)
  fields_flat = _sc_scatter(val.reshape(rows_total * 128),
                            idx.reshape(rows_total, 128), zeros)
  fields = fields_flat.reshape(_NX, _NY, _NZ)
  radii = rad.reshape(-1)[:n]
  return fields, radii


# speedup vs baseline: 1.0228x; 1.0228x over previous
import functools

import jax
import jax.numpy as jnp
from jax import lax
from jax.experimental import pallas as pl
from jax.experimental.pallas import tpu as pltpu
from jax.experimental.pallas import tpu_sc as plsc

_NX, _NY, _NZ = 128, 128, 128
_SX, _SY, _SZ = 2.0, 2.0, 2.0
_CX, _CY, _CZ = 0.0, 0.0, 0.0
_SCALE_MOD = 1.0
_HX, _HY, _HZ = _SX / _NX, _SY / _NY, _SZ / _NZ
_GX = _CX - _SX / 2 + 0.5 * _HX
_GY = _CY - _SY / 2 + 0.5 * _HY
_GZ = _CZ - _SZ / 2 + 0.5 * _HZ
_NVOX = _NX * _NY * _NZ
_HALF = _NVOX // 2

_BG_SUB = 32
_BG = _BG_SUB * 128
_OFF = (-1, 0, 1)


def _gauss_body(p_ref, val_ref, idx_ref, rad_ref):
  mx, my, mz = p_ref[0], p_ref[1], p_ref[2]
  sx = p_ref[3] * _SCALE_MOD
  sy = p_ref[4] * _SCALE_MOD
  sz = p_ref[5] * _SCALE_MOD
  qw, qx, qy, qz = p_ref[6], p_ref[7], p_ref[8], p_ref[9]
  op = p_ref[10]

  rmax = jnp.maximum(sx, jnp.maximum(sy, sz))
  rad_ref[...] = jnp.ceil(3.0 * rmax / (_SX / _NX)).astype(jnp.int32)

  nrm = jnp.sqrt(qw * qw + qx * qx + qy * qy + qz * qz) + 1e-8
  w, x, y, z = qw / nrm, qx / nrm, qy / nrm, qz / nrm
  r00 = 1 - 2 * (y * y + z * z)
  r01 = 2 * (x * y - w * z)
  r02 = 2 * (x * z + w * y)
  r10 = 2 * (x * y + w * z)
  r11 = 1 - 2 * (x * x + z * z)
  r12 = 2 * (y * z - w * x)
  r20 = 2 * (x * z - w * y)
  r21 = 2 * (y * z + w * x)
  r22 = 1 - 2 * (x * x + y * y)

  s0, s1, s2 = sx * sx, sy * sy, sz * sz
  a = r00 * r00 * s0 + r01 * r01 * s1 + r02 * r02 * s2 + 1e-8
  b = r00 * r10 * s0 + r01 * r11 * s1 + r02 * r12 * s2
  c = r00 * r20 * s0 + r01 * r21 * s1 + r02 * r22 * s2
  d = r10 * r10 * s0 + r11 * r11 * s1 + r12 * r12 * s2 + 1e-8
  e = r10 * r20 * s0 + r11 * r21 * s1 + r12 * r22 * s2
  f = r20 * r20 * s0 + r21 * r21 * s1 + r22 * r22 * s2 + 1e-8

  c00 = d * f - e * e
  c01 = c * e - b * f
  c02 = b * e - c * d
  c11 = a * f - c * c
  c12 = b * c - a * e
  c22 = a * d - b * b
  det = a * c00 + b * c01 + c * c02
  inv = 1.0 / det
  i00, i11, i22 = c00 * inv, c11 * inv, c22 * inv
  i01_2, i02_2, i12_2 = 2 * c01 * inv, 2 * c02 * inv, 2 * c12 * inv

  bxf = jnp.round((mx - _GX) / _HX)
  byf = jnp.round((my - _GY) / _HY)
  bzf = jnp.round((mz - _GZ) / _HZ)
  bxi = bxf.astype(jnp.int32)
  byi = byf.astype(jnp.int32)
  bzi = bzf.astype(jnp.int32)

  dxs, dys, dzs = {}, {}, {}
  vxm, vym, vzm = {}, {}, {}
  xc, yc, zc = {}, {}, {}
  for o in _OFF:
    of = float(o)
    dxs[o] = (_GX + (bxf + of) * _HX) - mx
    dys[o] = (_GY + (byf + of) * _HY) - my
    dzs[o] = (_GZ + (bzf + of) * _HZ) - mz
    ix, iy, iz = bxi + o, byi + o, bzi + o
    vxm[o] = (ix >= 0) & (ix < _NX)
    vym[o] = (iy >= 0) & (iy < _NY)
    vzm[o] = (iz >= 0) & (iz < _NZ)
    xc[o] = jnp.clip(ix, 0, _NX - 1) * (_NY * _NZ)
    yc[o] = jnp.clip(iy, 0, _NY - 1) * _NZ
    zc[o] = jnp.clip(iz, 0, _NZ - 1)

  i22dz = {o: i22 * dzs[o] for o in _OFF}
  k = 0
  for ox in _OFF:
    dx = dxs[ox]
    hx1 = i01_2 * dx
    hx2 = i02_2 * dx
    qx_part = i00 * dx * dx
    for oy in _OFF:
      dy = dys[oy]
      qxy = qx_part + dy * (i11 * dy + hx1)
      t2 = i12_2 * dy + hx2
      vxy = vxm[ox] & vym[oy]
      fxy = xc[ox] + yc[oy]
      for oz in _OFF:
        dz = dzs[oz]
        q = qxy + dz * (i22dz[oz] + t2)
        opv = jnp.where(vxy & vzm[oz], op, 0.0)
        alpha = opv * jnp.exp(-0.5 * q)
        flat = fxy + zc[oz]
        in0 = flat < _HALF
        val_ref[k] = alpha
        idx_ref[0, k] = jnp.where(in0, flat, -1)
        idx_ref[1, k] = jnp.where(in0, -1, flat - _HALF)
        k += 1


def _sc_scatter(valf, idxf, zeros):
  p_total = valf.shape[0]
  per_tile = p_total // 16
  win = 16128
  nwin = per_tile // win
  assert per_tile % win == 0
  seg = _HALF // 16
  mesh = plsc.VectorSubcoreMesh(core_axis_name="c", subcore_axis_name="s")

  @functools.partial(
      pl.kernel,
      mesh=mesh,
      out_type=jax.ShapeDtypeStruct((_NVOX,), jnp.float32),
      scratch_types=[
          pltpu.VMEM_SHARED((_HALF,), jnp.float32),
          pltpu.VMEM((win,), jnp.int32),
          pltpu.VMEM((win,), jnp.int32),
          pltpu.VMEM((win,), jnp.float32),
          pltpu.VMEM((win,), jnp.float32),
          pltpu.SemaphoreType.DMA,
          pltpu.SemaphoreType.DMA,
          pltpu.SemaphoreType.DMA,
          pltpu.SemaphoreType.DMA,
      ],
  )
  def scatter_kernel(val_hbm, idx_hbm, zero_hbm, out_hbm, grid_sp, idx_a,
                     idx_b, val_a, val_b, sem_la, sem_lb, sem_sa, sem_sb):
    cid = lax.axis_index("c")
    sid = lax.axis_index("s")
    pltpu.sync_copy(zero_hbm.at[pl.ds(sid * seg, seg)],
                    grid_sp.at[pl.ds(sid * seg, seg)])
    plsc.subcore_barrier()
    base = sid * per_tile

    idx_bufs = (idx_a, idx_b)
    val_bufs = (val_a, val_b)
    sem_l = (sem_la, sem_lb)
    sem_s = (sem_sa, sem_sb)

    def start_loads(w, bb):
      off = base + w * win
      hi = pltpu.async_copy(idx_hbm.at[cid, pl.ds(off, win)], idx_bufs[bb],
                            sem_l[bb])
      hv = pltpu.async_copy(val_hbm.at[pl.ds(off, win)], val_bufs[bb],
                            sem_l[bb])
      return hi, hv

    loads = [None, None]
    scats = [None, None]
    loads[0] = start_loads(0, 0)
    for wi in range(nwin):
      cur = wi & 1
      nxt = 1 - cur
      loads[cur][0].wait()
      loads[cur][1].wait()
      if wi + 1 < nwin:
        if scats[nxt] is not None:
          scats[nxt].wait()
        loads[nxt] = start_loads(wi + 1, nxt)
      scats[cur] = pltpu.async_copy(
          val_bufs[cur],
          grid_sp.at[plsc.Indices(idx_bufs[cur], ignored_value=-1)],
          sem_s[cur],
          add=True,
      )
    for bb in range(2):
      if scats[bb] is not None:
        scats[bb].wait()
    plsc.subcore_barrier()
    pltpu.sync_copy(grid_sp.at[pl.ds(sid * seg, seg)],
                    out_hbm.at[pl.ds(cid * _HALF + sid * seg, seg)])

  return scatter_kernel(valf, idxf, zeros)


def kernel(means3D, opacities, scales, rotations):
  n = means3D.shape[0]
  nb = -(-n // _BG)
  npad = nb * _BG
  sub = npad // 128
  pad = npad - n

  base = jnp.concatenate([means3D, scales, rotations, opacities], axis=1)
  fill = jnp.array(
      [0.0, 0.0, 0.0, 0.01, 0.01, 0.01, 1.0, 1.0, 1.0, 1.0, 0.0],
      jnp.float32)
  base = jnp.concatenate(
      [base, jnp.broadcast_to(fill, (pad, 11))], axis=0)
  params = base.T.reshape(11, sub, 128)

  val, idx, rad = pl.pallas_call(
      _gauss_body,
      grid=(nb,),
      in_specs=[pl.BlockSpec((11, _BG_SUB, 128), lambda i: (0, i, 0))],
      out_specs=[
          pl.BlockSpec((27, _BG_SUB, 128), lambda i: (0, i, 0)),
          pl.BlockSpec((2, 27, _BG_SUB, 128), lambda i: (0, 0, i, 0)),
          pl.BlockSpec((_BG_SUB, 128), lambda i: (i, 0)),
      ],
      out_shape=[
          jax.ShapeDtypeStruct((27, sub, 128), jnp.float32),
          jax.ShapeDtypeStruct((2, 27, sub, 128), jnp.int32),
          jax.ShapeDtypeStruct((sub, 128), jnp.int32),
      ],
  )(params)

  p_total = 27 * npad
  zeros = jnp.zeros((_HALF,), jnp.float32)
  fields_flat = _sc_scatter(val.reshape(p_total), idx.reshape(2, p_total),
                            zeros)
  fields = fields_flat.reshape(_NX, _NY, _NZ)
  radii = rad.reshape(-1)[:n]
  return fields, radii


# speedup vs baseline: 1.0332x; 1.0102x over previous
import functools

import jax
import jax.numpy as jnp
from jax import lax
from jax.experimental import pallas as pl
from jax.experimental.pallas import tpu as pltpu
from jax.experimental.pallas import tpu_sc as plsc

_NX, _NY, _NZ = 128, 128, 128
_SX, _SY, _SZ = 2.0, 2.0, 2.0
_CX, _CY, _CZ = 0.0, 0.0, 0.0
_SCALE_MOD = 1.0
_HX, _HY, _HZ = _SX / _NX, _SY / _NY, _SZ / _NZ
_GX = _CX - _SX / 2 + 0.5 * _HX
_GY = _CY - _SY / 2 + 0.5 * _HY
_GZ = _CZ - _SZ / 2 + 0.5 * _HZ
_NVOX = _NX * _NY * _NZ
_HALF = _NVOX // 2

_BG_SUB = 32
_BG = _BG_SUB * 128
_OFF = (-1, 0, 1)


def _gauss_body(p_ref, val_ref, idx_ref, rad_ref):
  mx, my, mz = p_ref[0], p_ref[1], p_ref[2]
  sx = p_ref[3] * _SCALE_MOD
  sy = p_ref[4] * _SCALE_MOD
  sz = p_ref[5] * _SCALE_MOD
  qw, qx, qy, qz = p_ref[6], p_ref[7], p_ref[8], p_ref[9]
  op = p_ref[10]

  rmax = jnp.maximum(sx, jnp.maximum(sy, sz))
  rad_ref[...] = jnp.ceil(3.0 * rmax / (_SX / _NX)).astype(jnp.int32)

  nrm = jnp.sqrt(qw * qw + qx * qx + qy * qy + qz * qz) + 1e-8
  w, x, y, z = qw / nrm, qx / nrm, qy / nrm, qz / nrm
  r00 = 1 - 2 * (y * y + z * z)
  r01 = 2 * (x * y - w * z)
  r02 = 2 * (x * z + w * y)
  r10 = 2 * (x * y + w * z)
  r11 = 1 - 2 * (x * x + z * z)
  r12 = 2 * (y * z - w * x)
  r20 = 2 * (x * z - w * y)
  r21 = 2 * (y * z + w * x)
  r22 = 1 - 2 * (x * x + y * y)

  s0, s1, s2 = sx * sx, sy * sy, sz * sz
  a = r00 * r00 * s0 + r01 * r01 * s1 + r02 * r02 * s2 + 1e-8
  b = r00 * r10 * s0 + r01 * r11 * s1 + r02 * r12 * s2
  c = r00 * r20 * s0 + r01 * r21 * s1 + r02 * r22 * s2
  d = r10 * r10 * s0 + r11 * r11 * s1 + r12 * r12 * s2 + 1e-8
  e = r10 * r20 * s0 + r11 * r21 * s1 + r12 * r22 * s2
  f = r20 * r20 * s0 + r21 * r21 * s1 + r22 * r22 * s2 + 1e-8

  c00 = d * f - e * e
  c01 = c * e - b * f
  c02 = b * e - c * d
  c11 = a * f - c * c
  c12 = b * c - a * e
  c22 = a * d - b * b
  det = a * c00 + b * c01 + c * c02
  inv = 1.0 / det
  i00, i11, i22 = c00 * inv, c11 * inv, c22 * inv
  i01_2, i02_2, i12_2 = 2 * c01 * inv, 2 * c02 * inv, 2 * c12 * inv

  bxf = jnp.round((mx - _GX) / _HX)
  byf = jnp.round((my - _GY) / _HY)
  bzf = jnp.round((mz - _GZ) / _HZ)
  bxi = bxf.astype(jnp.int32)
  byi = byf.astype(jnp.int32)
  bzi = bzf.astype(jnp.int32)

  dxs, dys, dzs = {}, {}, {}
  vxm, vym, vzm = {}, {}, {}
  xc, yc, zc = {}, {}, {}
  for o in _OFF:
    of = float(o)
    dxs[o] = (_GX + (bxf + of) * _HX) - mx
    dys[o] = (_GY + (byf + of) * _HY) - my
    dzs[o] = (_GZ + (bzf + of) * _HZ) - mz
    ix, iy, iz = bxi + o, byi + o, bzi + o
    vxm[o] = (ix >= 0) & (ix < _NX)
    vym[o] = (iy >= 0) & (iy < _NY)
    vzm[o] = (iz >= 0) & (iz < _NZ)
    xc[o] = jnp.clip(ix, 0, _NX - 1) * (_NY * _NZ)
    yc[o] = jnp.clip(iy, 0, _NY - 1) * _NZ
    zc[o] = jnp.clip(iz, 0, _NZ - 1)

  i22dz = {o: i22 * dzs[o] for o in _OFF}
  k = 0
  for ox in _OFF:
    dx = dxs[ox]
    hx1 = i01_2 * dx
    hx2 = i02_2 * dx
    qx_part = i00 * dx * dx
    for oy in _OFF:
      dy = dys[oy]
      qxy = qx_part + dy * (i11 * dy + hx1)
      t2 = i12_2 * dy + hx2
      vxy = vxm[ox] & vym[oy]
      fxy = xc[ox] + yc[oy]
      for oz in _OFF:
        dz = dzs[oz]
        q = qxy + dz * (i22dz[oz] + t2)
        opv = jnp.where(vxy & vzm[oz], op, 0.0)
        alpha = opv * jnp.exp(-0.5 * q)
        flat = fxy + zc[oz]
        in0 = flat < _HALF
        val_ref[k] = alpha
        idx_ref[0, k] = jnp.where(in0, flat, -1)
        idx_ref[1, k] = jnp.where(in0, -1, flat - _HALF)
        k += 1


def _sc_scatter(valf, idxf, zeros):
  p_total = valf.shape[0]
  per_tile = p_total // 16
  win = 16128
  nwin = per_tile // win
  assert per_tile % win == 0
  seg = _HALF // 16
  mesh = plsc.VectorSubcoreMesh(core_axis_name="c", subcore_axis_name="s")

  @functools.partial(
      pl.kernel,
      mesh=mesh,
      out_type=jax.ShapeDtypeStruct((_NVOX,), jnp.float32),
      scratch_types=[
          pltpu.VMEM_SHARED((_HALF,), jnp.float32),
          pltpu.VMEM((win,), jnp.int32),
          pltpu.VMEM((win,), jnp.int32),
          pltpu.VMEM((win,), jnp.float32),
          pltpu.VMEM((win,), jnp.float32),
          pltpu.SemaphoreType.DMA,
          pltpu.SemaphoreType.DMA,
          pltpu.SemaphoreType.DMA,
          pltpu.SemaphoreType.DMA,
      ],
  )
  def scatter_kernel(val_hbm, idx_hbm, zero_hbm, out_hbm, grid_sp, idx_a,
                     idx_b, val_a, val_b, sem_la, sem_lb, sem_sa, sem_sb):
    cid = lax.axis_index("c")
    sid = lax.axis_index("s")
    pltpu.sync_copy(zero_hbm.at[pl.ds(sid * seg, seg)],
                    grid_sp.at[pl.ds(sid * seg, seg)])
    plsc.subcore_barrier()
    base = sid * per_tile

    idx_bufs = (idx_a, idx_b)
    val_bufs = (val_a, val_b)
    sem_l = (sem_la, sem_lb)
    sem_s = (sem_sa, sem_sb)

    def start_loads(w, bb):
      off = base + w * win
      hi = pltpu.async_copy(idx_hbm.at[cid, pl.ds(off, win)], idx_bufs[bb],
                            sem_l[bb])
      hv = pltpu.async_copy(val_hbm.at[pl.ds(off, win)], val_bufs[bb],
                            sem_l[bb])
      return hi, hv

    loads = [None, None]
    scats = [None, None]
    loads[0] = start_loads(0, 0)
    for wi in range(nwin):
      cur = wi & 1
      nxt = 1 - cur
      loads[cur][0].wait()
      loads[cur][1].wait()
      if wi + 1 < nwin:
        if scats[nxt] is not None:
          scats[nxt].wait()
        loads[nxt] = start_loads(wi + 1, nxt)
      scats[cur] = pltpu.async_copy(
          val_bufs[cur],
          grid_sp.at[plsc.Indices(idx_bufs[cur], ignored_value=-1)],
          sem_s[cur],
          add=True,
      )
    for bb in range(2):
      if scats[bb] is not None:
        scats[bb].wait()
    plsc.subcore_barrier()
    pltpu.sync_copy(grid_sp.at[pl.ds(sid * seg, seg)],
                    out_hbm.at[pl.ds(cid * _HALF + sid * seg, seg)])

  return scatter_kernel(valf, idxf, zeros)


def kernel(means3D, opacities, scales, rotations):
  n = means3D.shape[0]
  nb = -(-n // _BG)
  npad = nb * _BG
  sub = npad // 128
  pad = npad - n

  def padn(arr, v):
    return jnp.pad(arr, ((0, 0), (0, pad)), constant_values=v)

  params = jnp.concatenate([
      padn(means3D.T, 0.0),
      padn(scales.T, 0.01),
      padn(rotations.T, 1.0),
      padn(opacities.T, 0.0),
  ], axis=0).reshape(11, sub, 128)

  val, idx, rad = pl.pallas_call(
      _gauss_body,
      grid=(nb,),
      in_specs=[pl.BlockSpec((11, _BG_SUB, 128), lambda i: (0, i, 0))],
      out_specs=[
          pl.BlockSpec((27, _BG_SUB, 128), lambda i: (0, i, 0)),
          pl.BlockSpec((2, 27, _BG_SUB, 128), lambda i: (0, 0, i, 0)),
          pl.BlockSpec((_BG_SUB, 128), lambda i: (i, 0)),
      ],
      out_shape=[
          jax.ShapeDtypeStruct((27, sub, 128), jnp.float32),
          jax.ShapeDtypeStruct((2, 27, sub, 128), jnp.int32),
          jax.ShapeDtypeStruct((sub, 128), jnp.int32),
      ],
  )(params)

  p_total = 27 * npad
  zeros = jnp.zeros((_HALF,), jnp.float32)
  fields_flat = _sc_scatter(val.reshape(p_total), idx.reshape(2, p_total),
                            zeros)
  fields = fields_flat.reshape(_NX, _NY, _NZ)
  radii = rad.reshape(-1)[:n]
  return fields, radii
